# initial kernel scaffold (unmeasured)
import jax
import jax.numpy as jnp
from jax import lax
from jax.experimental import pallas as pl
from jax.experimental.pallas import tpu as pltpu


def kernel(
    x,
):
    def body(*refs):
        pass

    out_shape = jax.ShapeDtypeStruct(..., jnp.float32)
    return pl.pallas_call(body, out_shape=out_shape)(...)


# baseline (device time: 798357 ns/iter reference)
import jax
import jax.numpy as jnp
from jax import lax
from jax.experimental import pallas as pl
from jax.experimental.pallas import tpu as pltpu

M, N = 16384, 1024
C = 8
CH = M // C


def kernel(x):
    def body(x_ref, out_ref, comm_ref, send_sem, recv_sem, ready_sem):
        i = pl.program_id(0)
        my_x = lax.axis_index("x")
        my_y = lax.axis_index("y")
        my_z = lax.axis_index("z")
        partner = (1 - my_x, my_y, my_z)

        @pl.when(i == 0)
        def _():
            barrier_sem = pltpu.get_barrier_semaphore()
            pl.semaphore_signal(
                barrier_sem, inc=1,
                device_id=partner, device_id_type=pl.DeviceIdType.MESH,
            )
            pl.semaphore_wait(barrier_sem, 1)

        @pl.when(i > 0)
        def _():
            pl.semaphore_wait(ready_sem, 1)

        rdma = pltpu.make_async_remote_copy(
            src_ref=x_ref,
            dst_ref=comm_ref,
            send_sem=send_sem,
            recv_sem=recv_sem,
            device_id=partner,
            device_id_type=pl.DeviceIdType.MESH,
        )
        rdma.start()
        rdma.wait()

        out_ref[:, :] = x_ref[:, :] + comm_ref[:, :]

        @pl.when(i < C - 1)
        def _():
            pl.semaphore_signal(
                ready_sem, inc=1,
                device_id=partner, device_id_type=pl.DeviceIdType.MESH,
            )

    return pl.pallas_call(
        body,
        grid=(C,),
        out_shape=jax.ShapeDtypeStruct((M, N), x.dtype),
        in_specs=[pl.BlockSpec((CH, N), lambda i: (i, 0))],
        out_specs=pl.BlockSpec((CH, N), lambda i: (i, 0)),
        scratch_shapes=[
            pltpu.VMEM((CH, N), x.dtype),
            pltpu.SemaphoreType.DMA,
            pltpu.SemaphoreType.DMA,
            pltpu.SemaphoreType.REGULAR,
        ],
        compiler_params=pltpu.CompilerParams(
            collective_id=0, vmem_limit_bytes=100 * 1024 * 1024
        ),
    )(x)


# device time: 475850 ns/iter; 1.6777x vs baseline; 1.6777x over previous
import jax
import jax.numpy as jnp
from jax import lax
from jax.experimental import pallas as pl
from jax.experimental.pallas import tpu as pltpu

M, N = 16384, 1024
C2 = 8
BLK = M // C2
CH = BLK // 2


def kernel(x):
    def body(x_ref, out_ref, xcomm, sumb, xsend, xrecv, ysend, yrecv,
             store_sem, x_ready, y_ready):
        j = pl.program_id(0)
        my_x = lax.axis_index("x")
        my_y = lax.axis_index("y")
        my_z = lax.axis_index("z")
        xp = (1 - my_x, my_y, my_z)
        yp = (my_x, 1 - my_y, my_z)

        half_off = my_y * CH
        rows_mine = (2 * j + my_y) * CH

        @pl.when(j == 0)
        def _():
            barrier = pltpu.get_barrier_semaphore()
            for nbr in (xp, yp):
                pl.semaphore_signal(barrier, inc=1, device_id=nbr,
                                    device_id_type=pl.DeviceIdType.MESH)
            pl.semaphore_wait(barrier, 2)

        def y_desc(slot, rows):
            return pltpu.make_async_remote_copy(
                src_ref=sumb.at[slot],
                dst_ref=out_ref.at[pl.ds(rows, CH)],
                send_sem=ysend.at[slot],
                recv_sem=yrecv.at[slot],
                device_id=yp,
                device_id_type=pl.DeviceIdType.MESH,
            )

        def step(s):
            sp = 1 - s

            @pl.when(j >= 2)
            def _():
                pl.semaphore_wait(x_ready, 1)

            x_rdma = pltpu.make_async_remote_copy(
                src_ref=x_ref.at[pl.ds(half_off, CH)],
                dst_ref=xcomm.at[s],
                send_sem=xsend.at[s],
                recv_sem=xrecv.at[s],
                device_id=xp,
                device_id_type=pl.DeviceIdType.MESH,
            )
            x_rdma.start()

            @pl.when(j >= 1)
            def _():
                rows_theirs_prev = (2 * (j - 1) + 1 - my_y) * CH
                y_desc(sp, rows_theirs_prev).wait_recv()

            @pl.when(jnp.logical_and(j >= 1, j <= C2 - 2))
            def _():
                pl.semaphore_signal(y_ready, inc=1, device_id=yp,
                                    device_id_type=pl.DeviceIdType.MESH)

            @pl.when(j >= 2)
            def _():
                old_rows = (2 * (j - 2) + my_y) * CH
                y_desc(s, old_rows).wait_send()
                pltpu.make_async_copy(
                    sumb.at[s], out_ref.at[pl.ds(old_rows, CH)],
                    store_sem.at[s],
                ).wait()

            x_rdma.wait()
            sumb[s, :, :] = x_ref[pl.ds(half_off, CH), :] + xcomm[s, :, :]

            @pl.when(j <= C2 - 3)
            def _():
                pl.semaphore_signal(x_ready, inc=1, device_id=xp,
                                    device_id_type=pl.DeviceIdType.MESH)

            store = pltpu.make_async_copy(
                sumb.at[s], out_ref.at[pl.ds(rows_mine, CH)], store_sem.at[s]
            )
            store.start()

            @pl.when(j >= 2)
            def _():
                pl.semaphore_wait(y_ready, 1)

            y_rdma = y_desc(s, rows_mine)
            y_rdma.start()

            @pl.when(j == C2 - 1)
            def _():
                rows_theirs = (2 * j + 1 - my_y) * CH
                y_desc(s, rows_theirs).wait_recv()
                y_rdma.wait_send()
                store.wait()
                prev_rows = (2 * (j - 1) + my_y) * CH
                y_desc(sp, prev_rows).wait_send()
                pltpu.make_async_copy(
                    sumb.at[sp], out_ref.at[pl.ds(prev_rows, CH)],
                    store_sem.at[sp],
                ).wait()

        @pl.when(j % 2 == 0)
        def _():
            step(0)

        @pl.when(j % 2 == 1)
        def _():
            step(1)

    return pl.pallas_call(
        body,
        grid=(C2,),
        out_shape=jax.ShapeDtypeStruct((M, N), x.dtype),
        in_specs=[pl.BlockSpec((BLK, N), lambda j: (j, 0))],
        out_specs=pl.BlockSpec(memory_space=pl.ANY),
        scratch_shapes=[
            pltpu.VMEM((2, CH, N), x.dtype),
            pltpu.VMEM((2, CH, N), x.dtype),
            pltpu.SemaphoreType.DMA((2,)),
            pltpu.SemaphoreType.DMA((2,)),
            pltpu.SemaphoreType.DMA((2,)),
            pltpu.SemaphoreType.DMA((2,)),
            pltpu.SemaphoreType.DMA((2,)),
            pltpu.SemaphoreType.REGULAR,
            pltpu.SemaphoreType.REGULAR,
        ],
        compiler_params=pltpu.CompilerParams(
            collective_id=0, vmem_limit_bytes=100 * 1024 * 1024
        ),
    )(x)


# device time: 360535 ns/iter; 2.2144x vs baseline; 1.3198x over previous
import jax
import jax.numpy as jnp
from jax import lax
from jax.experimental import pallas as pl
from jax.experimental.pallas import tpu as pltpu

M, N = 16384, 1024
C2 = 8
BLK = M // C2
QH = BLK // 4
QH2 = QH // 2

MESH = pl.DeviceIdType.MESH


def kernel(x):
    def body(x_ref, out_ref, xcomm, sm, yin, zin,
             xs, xr, yss, ysr, zss, zsr, yfs, yfr, zfs, zfr,
             stm, sty, stz, xrd, yrd, zrd, yfrd, zfrd):
        j = pl.program_id(0)
        my_x = lax.axis_index("x")
        my_y = lax.axis_index("y")
        my_z = lax.axis_index("z")
        zpar = my_z % 2
        xp = (1 - my_x, my_y, my_z)
        yp = (my_x, 1 - my_y, my_z)
        zp = (my_x, my_y, my_z - 2 * zpar + 1)

        tau = 2 * my_y + zpar
        tau_y = 2 * (1 - my_y) + zpar
        tau_z = 2 * my_y + 1 - zpar
        B = j * BLK

        @pl.when(j == 0)
        def _():
            barrier = pltpu.get_barrier_semaphore()
            for nbr in (xp, yp, zp):
                pl.semaphore_signal(barrier, inc=1, device_id=nbr,
                                    device_id_type=MESH)
            pl.semaphore_wait(barrier, 3)

        def step(s):
            sp = 1 - s
            Bp = (j - 1) * BLK

            @pl.when(j >= 2)
            def _():
                pl.semaphore_wait(xrd, 1)
            x_rdma = pltpu.make_async_remote_copy(
                src_ref=x_ref.at[pl.ds(tau * QH, QH)], dst_ref=xcomm.at[s],
                send_sem=xs.at[s], recv_sem=xr.at[s],
                device_id=xp, device_id_type=MESH)
            x_rdma.start()

            @pl.when(j >= 2)
            def _():
                pltpu.make_async_remote_copy(
                    src_ref=zin.at[s].at[pl.ds(0, QH2)],
                    dst_ref=out_ref.at[pl.ds((j - 2) * BLK + tau_z * QH, QH2)],
                    send_sem=yfs.at[sp], recv_sem=yfr.at[sp],
                    device_id=yp, device_id_type=MESH).wait_recv()
                pltpu.make_async_remote_copy(
                    src_ref=yin.at[s].at[pl.ds(QH2, QH2)],
                    dst_ref=out_ref.at[
                        pl.ds((j - 2) * BLK + tau_y * QH + QH2, QH2)],
                    send_sem=zfs.at[sp], recv_sem=zfr.at[sp],
                    device_id=zp, device_id_type=MESH).wait_recv()

            @pl.when(jnp.logical_and(j >= 2, j <= C2 - 2))
            def _():
                pl.semaphore_signal(yfrd, inc=1, device_id=yp,
                                    device_id_type=MESH)
                pl.semaphore_signal(zfrd, inc=1, device_id=zp,
                                    device_id_type=MESH)

            @pl.when(j >= 1)
            def _():
                pltpu.make_async_remote_copy(
                    src_ref=sm.at[sp], dst_ref=zin.at[sp],
                    send_sem=zss.at[sp], recv_sem=zsr.at[sp],
                    device_id=zp, device_id_type=MESH).wait_recv()

                @pl.when(j >= 3)
                def _():
                    pl.semaphore_wait(yfrd, 1)
                f4 = pltpu.make_async_remote_copy(
                    src_ref=zin.at[sp].at[pl.ds(0, QH2)],
                    dst_ref=out_ref.at[pl.ds(Bp + tau_z * QH, QH2)],
                    send_sem=yfs.at[s], recv_sem=yfr.at[s],
                    device_id=yp, device_id_type=MESH)
                f4.start()

                pltpu.make_async_remote_copy(
                    src_ref=sm.at[sp], dst_ref=yin.at[sp],
                    send_sem=yss.at[sp], recv_sem=ysr.at[sp],
                    device_id=yp, device_id_type=MESH).wait_recv()

                @pl.when(j >= 3)
                def _():
                    pl.semaphore_wait(zfrd, 1)
                f5 = pltpu.make_async_remote_copy(
                    src_ref=yin.at[sp].at[pl.ds(QH2, QH2)],
                    dst_ref=out_ref.at[pl.ds(Bp + tau_y * QH + QH2, QH2)],
                    send_sem=zfs.at[s], recv_sem=zfr.at[s],
                    device_id=zp, device_id_type=MESH)
                f5.start()

                pltpu.make_async_copy(
                    zin.at[sp], out_ref.at[pl.ds(Bp + tau_z * QH, QH)],
                    stz.at[sp]).start()
                pltpu.make_async_copy(
                    yin.at[sp], out_ref.at[pl.ds(Bp + tau_y * QH, QH)],
                    sty.at[sp]).start()

            x_rdma.wait_recv()

            @pl.when(j >= 2)
            def _():
                pltpu.make_async_remote_copy(
                    src_ref=sm.at[s], dst_ref=yin.at[s],
                    send_sem=yss.at[s], recv_sem=ysr.at[s],
                    device_id=yp, device_id_type=MESH).wait_send()
                pltpu.make_async_remote_copy(
                    src_ref=sm.at[s], dst_ref=zin.at[s],
                    send_sem=zss.at[s], recv_sem=zsr.at[s],
                    device_id=zp, device_id_type=MESH).wait_send()
                pltpu.make_async_copy(
                    sm.at[s], out_ref.at[pl.ds(B + tau * QH, QH)],
                    stm.at[s]).wait()

            sm[s, :, :] = x_ref[pl.ds(tau * QH, QH), :] + xcomm[s, :, :]

            @pl.when(j <= C2 - 3)
            def _():
                pl.semaphore_signal(xrd, inc=1, device_id=xp,
                                    device_id_type=MESH)
            x_rdma.wait_send()

            @pl.when(j >= 2)
            def _():
                pl.semaphore_wait(yrd, 1)
            f2 = pltpu.make_async_remote_copy(
                src_ref=sm.at[s], dst_ref=yin.at[s],
                send_sem=yss.at[s], recv_sem=ysr.at[s],
                device_id=yp, device_id_type=MESH)
            f2.start()

            @pl.when(j >= 2)
            def _():
                pl.semaphore_wait(zrd, 1)
            f3 = pltpu.make_async_remote_copy(
                src_ref=sm.at[s], dst_ref=zin.at[s],
                send_sem=zss.at[s], recv_sem=zsr.at[s],
                device_id=zp, device_id_type=MESH)
            f3.start()

            pltpu.make_async_copy(
                sm.at[s], out_ref.at[pl.ds(B + tau * QH, QH)],
                stm.at[s]).start()

            @pl.when(j >= 1)
            def _():
                pltpu.make_async_copy(
                    zin.at[sp], out_ref.at[pl.ds(Bp + tau_z * QH, QH)],
                    stz.at[sp]).wait()
                pltpu.make_async_remote_copy(
                    src_ref=zin.at[sp].at[pl.ds(0, QH2)],
                    dst_ref=out_ref.at[pl.ds(Bp + tau_z * QH, QH2)],
                    send_sem=yfs.at[s], recv_sem=yfr.at[s],
                    device_id=yp, device_id_type=MESH).wait_send()
                pltpu.make_async_copy(
                    yin.at[sp], out_ref.at[pl.ds(Bp + tau_y * QH, QH)],
                    sty.at[sp]).wait()
                pltpu.make_async_remote_copy(
                    src_ref=yin.at[sp].at[pl.ds(QH2, QH2)],
                    dst_ref=out_ref.at[pl.ds(Bp + tau_y * QH + QH2, QH2)],
                    send_sem=zfs.at[s], recv_sem=zfr.at[s],
                    device_id=zp, device_id_type=MESH).wait_send()

            @pl.when(jnp.logical_and(j >= 1, j <= C2 - 2))
            def _():
                pl.semaphore_signal(zrd, inc=1, device_id=zp,
                                    device_id_type=MESH)
                pl.semaphore_signal(yrd, inc=1, device_id=yp,
                                    device_id_type=MESH)

            @pl.when(j == C2 - 1)
            def _():
                Bl = B
                pltpu.make_async_remote_copy(
                    src_ref=sm.at[s], dst_ref=zin.at[s],
                    send_sem=zss.at[s], recv_sem=zsr.at[s],
                    device_id=zp, device_id_type=MESH).wait_recv()
                f4d = pltpu.make_async_remote_copy(
                    src_ref=zin.at[s].at[pl.ds(0, QH2)],
                    dst_ref=out_ref.at[pl.ds(Bl + tau_z * QH, QH2)],
                    send_sem=yfs.at[sp], recv_sem=yfr.at[sp],
                    device_id=yp, device_id_type=MESH)
                f4d.start()
                pltpu.make_async_remote_copy(
                    src_ref=sm.at[s], dst_ref=yin.at[s],
                    send_sem=yss.at[s], recv_sem=ysr.at[s],
                    device_id=yp, device_id_type=MESH).wait_recv()
                f5d = pltpu.make_async_remote_copy(
                    src_ref=yin.at[s].at[pl.ds(QH2, QH2)],
                    dst_ref=out_ref.at[pl.ds(Bl + tau_y * QH + QH2, QH2)],
                    send_sem=zfs.at[sp], recv_sem=zfr.at[sp],
                    device_id=zp, device_id_type=MESH)
                f5d.start()
                pltpu.make_async_copy(
                    zin.at[s], out_ref.at[pl.ds(Bl + tau_z * QH, QH)],
                    stz.at[s]).start()
                pltpu.make_async_copy(
                    yin.at[s], out_ref.at[pl.ds(Bl + tau_y * QH, QH)],
                    sty.at[s]).start()
                pltpu.make_async_remote_copy(
                    src_ref=zin.at[sp].at[pl.ds(0, QH2)],
                    dst_ref=out_ref.at[pl.ds(Bp + tau_z * QH, QH2)],
                    send_sem=yfs.at[s], recv_sem=yfr.at[s],
                    device_id=yp, device_id_type=MESH).wait_recv()
                pltpu.make_async_remote_copy(
                    src_ref=yin.at[sp].at[pl.ds(QH2, QH2)],
                    dst_ref=out_ref.at[pl.ds(Bp + tau_y * QH + QH2, QH2)],
                    send_sem=zfs.at[s], recv_sem=zfr.at[s],
                    device_id=zp, device_id_type=MESH).wait_recv()
                f4d.wait_recv()
                f5d.wait_recv()
                f4d.wait_send()
                f5d.wait_send()
                pltpu.make_async_copy(
                    zin.at[s], out_ref.at[pl.ds(Bl + tau_z * QH, QH)],
                    stz.at[s]).wait()
                pltpu.make_async_copy(
                    yin.at[s], out_ref.at[pl.ds(Bl + tau_y * QH, QH)],
                    sty.at[s]).wait()
                f2.wait_send()
                f3.wait_send()
                pltpu.make_async_copy(
                    sm.at[s], out_ref.at[pl.ds(Bl + tau * QH, QH)],
                    stm.at[s]).wait()
                pltpu.make_async_remote_copy(
                    src_ref=sm.at[sp], dst_ref=yin.at[sp],
                    send_sem=yss.at[sp], recv_sem=ysr.at[sp],
                    device_id=yp, device_id_type=MESH).wait_send()
                pltpu.make_async_remote_copy(
                    src_ref=sm.at[sp], dst_ref=zin.at[sp],
                    send_sem=zss.at[sp], recv_sem=zsr.at[sp],
                    device_id=zp, device_id_type=MESH).wait_send()
                pltpu.make_async_copy(
                    sm.at[sp], out_ref.at[pl.ds(Bp + tau * QH, QH)],
                    stm.at[sp]).wait()

        @pl.when(j % 2 == 0)
        def _():
            step(0)

        @pl.when(j % 2 == 1)
        def _():
            step(1)

    return pl.pallas_call(
        body,
        grid=(C2,),
        out_shape=jax.ShapeDtypeStruct((M, N), x.dtype),
        in_specs=[pl.BlockSpec((BLK, N), lambda j: (j, 0))],
        out_specs=pl.BlockSpec(memory_space=pl.ANY),
        scratch_shapes=[
            pltpu.VMEM((2, QH, N), x.dtype),
            pltpu.VMEM((2, QH, N), x.dtype),
            pltpu.VMEM((2, QH, N), x.dtype),
            pltpu.VMEM((2, QH, N), x.dtype),
            pltpu.SemaphoreType.DMA((2,)),
            pltpu.SemaphoreType.DMA((2,)),
            pltpu.SemaphoreType.DMA((2,)),
            pltpu.SemaphoreType.DMA((2,)),
            pltpu.SemaphoreType.DMA((2,)),
            pltpu.SemaphoreType.DMA((2,)),
            pltpu.SemaphoreType.DMA((2,)),
            pltpu.SemaphoreType.DMA((2,)),
            pltpu.SemaphoreType.DMA((2,)),
            pltpu.SemaphoreType.DMA((2,)),
            pltpu.SemaphoreType.DMA((2,)),
            pltpu.SemaphoreType.DMA((2,)),
            pltpu.SemaphoreType.DMA((2,)),
            pltpu.SemaphoreType.REGULAR,
            pltpu.SemaphoreType.REGULAR,
            pltpu.SemaphoreType.REGULAR,
            pltpu.SemaphoreType.REGULAR,
            pltpu.SemaphoreType.REGULAR,
        ],
        compiler_params=pltpu.CompilerParams(
            collective_id=0, vmem_limit_bytes=100 * 1024 * 1024
        ),
    )(x)
